# Initial kernel scaffold; baseline (speedup 1.0000x reference)
import jax
import jax.numpy as jnp
from jax.experimental import pallas as pl


def kernel(pos, latents):
    raise NotImplementedError("write your pallas kernel here")


# speedup vs baseline: 8.5381x; 8.5381x over previous
import functools
import math

import jax
import jax.numpy as jnp
from jax import lax
from jax.experimental import pallas as pl
from jax.experimental.pallas import tpu as pltpu
from jax.experimental.pallas import tpu_sc as plsc

DIM = 3
LVLS = 16
T = 524288
N_MIN = 16
N_MAX = 2048
F = 2


def _is_prime(n):
    if n < 2:
        return False
    if n % 2 == 0:
        return n == 2
    i = 3
    while i * i <= n:
        if n % i == 0:
            return False
        i += 2
    return True


def _next_prime(n):
    while not _is_prime(n):
        n += 1
    return n


P1 = _next_prime(1 << 17)
P2 = _next_prime(1 << 18)

_b = math.exp((math.log(N_MAX) - math.log(N_MIN)) / (LVLS - 1))
RES = []
METH = []
OFF = [0]
for _i in range(LVLS):
    _r = int(N_MIN * _b ** _i)
    RES.append(_r)
    _ne = (_r + 1) ** 2
    if _ne <= T:
        METH.append("one")
    else:
        METH.append("hash")
        _ne = T
    OFF.append(OFF[-1] + _ne)
ROWS = OFF[-1]
RMAX = ROWS - 1
MASK = T - 1

NPTS = 131072
NW = 32
PW = NPTS // NW
C = 128
NCH = PW // C
NG = C // 16
NIDX = LVLS * 8 * C
NFLT = 2 * NIDX
GB = 128
NDMA = NFLT // GB


@functools.cache
def _build_encoder():
    mesh = plsc.VectorSubcoreMesh(core_axis_name="c", subcore_axis_name="s")

    @functools.partial(
        pl.kernel,
        out_type=jax.ShapeDtypeStruct((NPTS * 2 * LVLS,), jnp.float32),
        mesh=mesh,
        scratch_types=[
            pltpu.VMEM((C,), jnp.float32),
            pltpu.VMEM((C,), jnp.float32),
            pltpu.VMEM((C,), jnp.float32),
            pltpu.VMEM((NFLT,), jnp.int32),
            pltpu.VMEM((NIDX,), jnp.float32),
            pltpu.VMEM((NFLT,), jnp.float32),
            pltpu.VMEM((C * 2 * LVLS,), jnp.float32),
            pltpu.SemaphoreType.DMA,
        ],
    )
    def _encode(px_hbm, py_hbm, pz_hbm, latf_hbm, out_hbm,
                pxb, pyb, pzb, idxb, wb, rowsb, outb, sem):
        wid = lax.axis_index("s") * 2 + lax.axis_index("c")
        iota = lax.iota(jnp.int32, 16)
        zf = jnp.zeros((16,), jnp.float32)

        def chunk_body(t, carry):
            base = wid * PW + t * C
            pltpu.sync_copy(px_hbm.at[pl.ds(base, C)], pxb)
            pltpu.sync_copy(py_hbm.at[pl.ds(base, C)], pyb)
            pltpu.sync_copy(pz_hbm.at[pl.ds(base, C)], pzb)

            def compute_group(g, c2):
                g16 = g * 16
                x = pxb[pl.ds(g16, 16)]
                y = pyb[pl.ds(g16, 16)]
                z = pzb[pl.ds(g16, 16)]
                for l in range(LVLS):
                    res = RES[l]
                    off = OFF[l]
                    sx = x * jnp.float32(res)
                    sy = y * jnp.float32(res)
                    sz = z * jnp.float32(res)
                    ix = sx.astype(jnp.int32)
                    iy = sy.astype(jnp.int32)
                    iz = sz.astype(jnp.int32)
                    fx = sx - ix.astype(jnp.float32)
                    fy = sy - iy.astype(jnp.float32)
                    fz = sz - iz.astype(jnp.float32)
                    gx = 1.0 - fx
                    gy = 1.0 - fy
                    gz = 1.0 - fz
                    wxy = (gx * gy, gx * fy, fx * gy, fx * fy)
                    wz = (gz, fz)
                    if METH[l] == "one":
                        r2 = res * res
                        t0 = (ix + res) * r2 + off
                        ts = (t0, t0 + r2)
                        u0 = iy * res
                        us = (u0, u0 + res)
                        vs = (iz, iz + 1)
                    else:
                        ts = (ix, ix + 1)
                        hy0 = iy * P1
                        us = (hy0, hy0 + P1)
                        hz0 = iz * P2
                        vs = (hz0, hz0 + P2)
                    for dx in range(2):
                        for dy in range(2):
                            for dz in range(2):
                                cc = dx * 4 + dy * 2 + dz
                                if METH[l] == "one":
                                    idx = jnp.minimum(
                                        ts[dx] + us[dy] + vs[dz], RMAX)
                                else:
                                    idx = (
                                        (ts[dx] ^ us[dy] ^ vs[dz]) & MASK
                                    ) + off
                                e0 = idx + idx
                                lc = l * 8 + cc
                                idxb[pl.ds(lc * 2 * C + g16, 16)] = e0
                                idxb[pl.ds(lc * 2 * C + C + g16, 16)] = e0 + 1
                                wb[pl.ds(lc * C + g16, 16)] = (
                                    wxy[dx * 2 + dy] * wz[dz])
                return c2

            lax.fori_loop(0, NG, compute_group, 0)

            def fire(j, c2):
                pltpu.make_async_copy(
                    latf_hbm.at[idxb.at[pl.ds(j * GB, GB)]],
                    rowsb.at[pl.ds(j * GB, GB)],
                    sem,
                ).start()
                return c2

            lax.fori_loop(0, NDMA, fire, 0)

            def drain(j, c2):
                pltpu.make_async_copy(
                    latf_hbm.at[idxb.at[pl.ds(j * GB, GB)]],
                    rowsb.at[pl.ds(j * GB, GB)],
                    sem,
                ).wait()
                return c2

            lax.fori_loop(0, NDMA, drain, 0)

            def accum_group(g, c2):
                g16 = g * 16
                for l in range(LVLS):
                    a0 = zf
                    a1 = zf
                    for cc in range(8):
                        lc = l * 8 + cc
                        w = wb[pl.ds(lc * C + g16, 16)]
                        r0 = rowsb[pl.ds(lc * 2 * C + g16, 16)]
                        r1 = rowsb[pl.ds(lc * 2 * C + C + g16, 16)]
                        a0 = a0 + w * r0
                        a1 = a1 + w * r1
                    outb[pl.ds((2 * l) * C + g16, 16)] = a0
                    outb[pl.ds((2 * l + 1) * C + g16, 16)] = a1
                return c2

            lax.fori_loop(0, NG, accum_group, 0)

            q = wid * NCH + t
            pltpu.sync_copy(outb, out_hbm.at[pl.ds(q * 2 * LVLS * C, 2 * LVLS * C)])
            return carry

        lax.fori_loop(0, NCH, chunk_body, 0)

    return _encode


def kernel(pos, latents):
    px = pos[:, 0]
    py = pos[:, 1]
    pz = pos[:, 2]
    latf = jnp.reshape(latents, (-1,))
    flat = _build_encoder()(px, py, pz, latf)
    cube = jnp.reshape(flat, (NPTS // C, 2 * LVLS, C))
    return jnp.reshape(jnp.transpose(cube, (0, 2, 1)), (NPTS, 2 * LVLS))
